# baseline (speedup 1.0000x reference)
import functools

import jax
import jax.numpy as jnp
from jax import lax
from jax.experimental import pallas as pl
from jax.experimental.pallas import tpu as pltpu
from jax.experimental.pallas import tpu_sc as plsc


def _make_gather(V, D, BATCH, SEQ):
    info = plsc.get_sparse_core_info()
    NC, NS = info.num_cores, info.num_subcores
    NW = NC * NS
    B = BATCH * SEQ
    assert B % NW == 0
    b_per_w = B // NW
    assert SEQ % b_per_w == 0
    C = 16
    NBUF = 4
    n_chunks = b_per_w // C
    n_rounds = n_chunks // NBUF
    assert b_per_w % (C * NBUF) == 0

    mesh = plsc.VectorSubcoreMesh(core_axis_name="c", subcore_axis_name="s")

    @functools.partial(
        pl.kernel,
        mesh=mesh,
        out_type=jax.ShapeDtypeStruct((BATCH, SEQ, D), jnp.float32),
        scratch_types=[
            pltpu.VMEM((b_per_w,), jnp.int32),
        ]
        + [pltpu.VMEM((C, D), jnp.float32) for _ in range(NBUF)]
        + [pltpu.SemaphoreType.DMA for _ in range(2 * NBUF)],
    )
    def gather_kernel(idx_hbm, table_hbm, out_hbm, idx_v, *rest):
        bufs = rest[:NBUF]
        gsems = rest[NBUF : 2 * NBUF]
        ssems = rest[2 * NBUF :]
        wid = lax.axis_index("s") * NC + lax.axis_index("c")
        base = wid * b_per_w
        bat = base // SEQ
        s_off = base % SEQ
        pltpu.sync_copy(idx_hbm.at[bat, pl.ds(s_off, b_per_w)], idx_v)

        def start_gather(g, b):
            pltpu.async_copy(
                table_hbm.at[idx_v.at[pl.ds(g * C, C)]], bufs[b], gsems[b]
            )

        def wait_gather(b):
            pltpu.make_async_copy(
                table_hbm.at[idx_v.at[pl.ds(0, C)]], bufs[b], gsems[b]
            ).wait()

        def start_scatter(g, b):
            pltpu.async_copy(
                bufs[b], out_hbm.at[bat, pl.ds(s_off + g * C, C)], ssems[b]
            )

        def wait_scatter(b):
            pltpu.make_async_copy(
                bufs[b], out_hbm.at[bat, pl.ds(s_off, C)], ssems[b]
            ).wait()

        start_gather(0, 0)
        start_gather(1, 1)

        def body(s, carry):
            for b in range(NBUF):
                g = s * NBUF + b
                wait_gather(b)
                start_scatter(g, b)
                nxt = (b + 2) % NBUF

                @pl.when(g + 2 >= NBUF)
                def _():
                    wait_scatter(nxt)

                @pl.when(g + 2 < n_chunks)
                def _():
                    start_gather(g + 2, nxt)

            return carry

        lax.fori_loop(0, n_rounds, body, 0)
        wait_scatter((n_chunks - 2) % NBUF)
        wait_scatter((n_chunks - 1) % NBUF)

    return gather_kernel


def kernel(position_ids, pos_embed):
    b, s = position_ids.shape
    v, d = pos_embed.shape
    return _make_gather(v, d, b, s)(position_ids, pos_embed)


# speedup vs baseline: 1.0049x; 1.0049x over previous
import functools

import jax
import jax.numpy as jnp
from jax import lax
from jax.experimental import pallas as pl
from jax.experimental.pallas import tpu as pltpu
from jax.experimental.pallas import tpu_sc as plsc


def _make_gather(V, D, BATCH, SEQ):
    info = plsc.get_sparse_core_info()
    NC, NS = info.num_cores, info.num_subcores
    NW = NC * NS
    B = BATCH * SEQ
    assert B % NW == 0
    b_per_w = B // NW
    assert SEQ % b_per_w == 0
    C = 8
    NBUF = 8
    DEPTH = NBUF // 2
    n_chunks = b_per_w // C
    n_rounds = n_chunks // NBUF
    assert b_per_w % (C * NBUF) == 0

    mesh = plsc.VectorSubcoreMesh(core_axis_name="c", subcore_axis_name="s")

    @functools.partial(
        pl.kernel,
        mesh=mesh,
        out_type=jax.ShapeDtypeStruct((BATCH, SEQ, D), jnp.float32),
        scratch_types=[
            pltpu.VMEM((b_per_w,), jnp.int32),
        ]
        + [pltpu.VMEM((C, D), jnp.float32) for _ in range(NBUF)]
        + [pltpu.SemaphoreType.DMA for _ in range(2 * NBUF)],
    )
    def gather_kernel(idx_hbm, table_hbm, out_hbm, idx_v, *rest):
        bufs = rest[:NBUF]
        gsems = rest[NBUF : 2 * NBUF]
        ssems = rest[2 * NBUF :]
        wid = lax.axis_index("s") * NC + lax.axis_index("c")
        base = wid * b_per_w
        bat = base // SEQ
        s_off = base % SEQ
        pltpu.sync_copy(idx_hbm.at[bat, pl.ds(s_off, b_per_w)], idx_v)

        def start_gather(g, b):
            pltpu.async_copy(
                table_hbm.at[idx_v.at[pl.ds(g * C, C)]], bufs[b], gsems[b]
            )

        def wait_gather(b):
            pltpu.make_async_copy(
                table_hbm.at[idx_v.at[pl.ds(0, C)]], bufs[b], gsems[b]
            ).wait()

        def start_scatter(g, b):
            pltpu.async_copy(
                bufs[b], out_hbm.at[bat, pl.ds(s_off + g * C, C)], ssems[b]
            )

        def wait_scatter(b):
            pltpu.make_async_copy(
                bufs[b], out_hbm.at[bat, pl.ds(s_off, C)], ssems[b]
            ).wait()

        for d in range(DEPTH):
            start_gather(d, d)

        def body(s, carry):
            for b in range(NBUF):
                g = s * NBUF + b
                wait_gather(b)
                start_scatter(g, b)
                nxt = (b + DEPTH) % NBUF

                @pl.when(g + DEPTH >= NBUF)
                def _():
                    wait_scatter(nxt)

                @pl.when(g + DEPTH < n_chunks)
                def _():
                    start_gather(g + DEPTH, nxt)

            return carry

        lax.fori_loop(0, n_rounds, body, 0)
        for d in range(DEPTH):
            wait_scatter((n_chunks - DEPTH + d) % NBUF)

    return gather_kernel


def kernel(position_ids, pos_embed):
    b, s = position_ids.shape
    v, d = pos_embed.shape
    return _make_gather(v, d, b, s)(position_ids, pos_embed)
